# Initial kernel scaffold; baseline (speedup 1.0000x reference)
import jax
import jax.numpy as jnp
from jax.experimental import pallas as pl


def kernel(x, w):
    raise NotImplementedError("write your pallas kernel here")


# speedup vs baseline: 4.4801x; 4.4801x over previous
import jax
import jax.numpy as jnp
from jax import lax
from jax.experimental import pallas as pl
from jax.experimental.pallas import tpu as pltpu

N = 4096
D = 64
ROWS = 32
LANES = 128


def _warp_body(x_ref, w_ref, out_ref):
    x = x_ref[...]
    w = w_ref[...]
    t1 = jnp.dot(x, w, preferred_element_type=jnp.float32)
    a_col = 1.0 / (1.0 + jnp.exp(-t1))

    ridx = lax.broadcasted_iota(jnp.int32, (N, LANES), 0)
    lidx = lax.broadcasted_iota(jnp.int32, (N, LANES), 1)
    Mm = jnp.where((ridx & (LANES - 1)) == lidx, 1.0, 0.0)
    am = jnp.sum((a_col * Mm).reshape(ROWS, LANES, LANES), axis=1)

    b = am
    lane = lax.broadcasted_iota(jnp.int32, (ROWS, LANES), 1)
    for sh in (1, 2, 4, 8, 16, 32, 64):
        b = b + jnp.where(lane >= sh, pltpu.roll(b, sh, axis=1), 0.0)
    tot = b[:, LANES - 1:LANES]
    p = tot
    row = lax.broadcasted_iota(jnp.int32, (ROWS, 1), 0)
    for sh in (1, 2, 4, 8, 16):
        p = p + jnp.where(row >= sh, pltpu.roll(p, sh, axis=0), 0.0)
    b = b + (p - tot)

    bR = jnp.broadcast_to(b[:, None, :], (ROWS, LANES, LANES)
                          ).reshape(N, LANES)
    b_col = jnp.sum(bR * Mm, axis=1, keepdims=True)

    ind_col = jnp.floor(b_col)
    prev_col = jnp.floor(b_col - a_col)
    cross_col = ind_col != prev_col
    frac_col = b_col - ind_col
    vm_col = jnp.where(cross_col, frac_col, a_col)
    ve_col = a_col - vm_col
    indi = ind_col.astype(jnp.int32)

    rlane = lax.broadcasted_iota(jnp.int32, (N, N), 1)
    pm = jnp.where(indi == rlane, 1.0, 0.0)
    pe = jnp.where(indi - 1 == rlane, 1.0, 0.0)

    ym = vm_col * x
    ye = ve_col * x
    contract = (((0,), (0,)), ((), ()))
    out = lax.dot_general(pm, ym, contract,
                          preferred_element_type=jnp.float32)
    out = out + lax.dot_general(pe, ye, contract,
                                preferred_element_type=jnp.float32)
    out_ref[...] = out


_warp = pl.pallas_call(
    _warp_body,
    out_shape=jax.ShapeDtypeStruct((N, D), jnp.float32),
)


@jax.jit
def kernel(x, w):
    return _warp(x, w)
